# baseline (speedup 1.0000x reference)
import math

import jax
import jax.numpy as jnp
from jax import lax
from jax.experimental import pallas as pl
from jax.experimental.pallas import tpu as pltpu
from jax.experimental.pallas import tpu_sc as plsc

D_MODEL = 64
SCALE = math.sqrt(D_MODEL)
OUT_W = 128

NC = 2
NS = 16
NW = NC * NS
L = 16

BLK = 128
R = 8


def _encoder_body(idx_hbm, wt_hbm, out_hbm, idx_v, out_v, *slabs_sems):
    slabs = slabs_sems[:R]
    sems = slabs_sems[R:2 * R]
    osem = slabs_sems[2 * R]
    n_per_w = idx_hbm.shape[1]
    wid = lax.axis_index("s") * NC + lax.axis_index("c")

    pltpu.sync_copy(idx_hbm.at[wid], idx_v.at[pl.ds(0, n_per_w)])

    iota = lax.iota(jnp.int32, L)

    def fire(t, r):
        blk0 = pl.multiple_of((t >> 7) << 7, BLK)
        half = D_MODEL // 2
        for h in range(2):
            pltpu.async_copy(
                wt_hbm.at[pl.ds(h * half, half), pl.ds(blk0, BLK)],
                slabs[r].at[pl.ds(h * half, half)],
                sems[r],
            )

    def extract(t, i, r):
        j_vec = jnp.full((L,), t & (BLK - 1), jnp.int32)
        for c0 in range(0, D_MODEL, L):
            vals = plsc.load_gather(slabs[r], [c0 + iota, j_vec])
            out_v[i, pl.ds(c0, L)] = vals * SCALE

    t_prime = idx_v[pl.ds(0, L)]
    for r in range(R):
        fire(t_prime[r], r)

    def loop_body(g, carry):
        base = g * L
        t_cur = idx_v[pl.ds(base, L)]
        t_nxt = idx_v[pl.ds(base + L, L)]
        for k in range(L):
            i = base + k
            r = k % R
            pltpu.make_async_copy(
                wt_hbm.at[:, pl.ds(0, BLK)], slabs[r], sems[r]
            ).wait()
            extract(t_cur[k], i, r)
            t_ahead = t_cur[k + R] if k + R < L else t_nxt[k + R - L]

            @pl.when(i + R < n_per_w)
            def _():
                fire(t_ahead, r)

        return carry

    lax.fori_loop(0, n_per_w // L, loop_body, 0)

    pltpu.async_copy(
        out_v, out_hbm.at[pl.ds(wid * n_per_w, n_per_w)], osem
    ).wait()


def kernel(src, weight):
    seq_len, batch = src.shape
    n_tokens, d_model = weight.shape
    b_total = seq_len * batch
    assert d_model == D_MODEL
    n_per_w = b_total // NW
    assert b_total % NW == 0 and n_per_w % R == 0

    idx = src.reshape(NW, n_per_w).astype(jnp.int32)
    wt = weight.T

    gather = pl.kernel(
        _encoder_body,
        out_type=jax.ShapeDtypeStruct((b_total, OUT_W), jnp.float32),
        mesh=plsc.VectorSubcoreMesh(
            core_axis_name="c", subcore_axis_name="s",
            num_cores=NC, num_subcores=NS,
        ),
        scratch_types=(
            [
                pltpu.VMEM((n_per_w + L,), jnp.int32),
                pltpu.VMEM((n_per_w, OUT_W), jnp.float32),
            ]
            + [pltpu.VMEM((D_MODEL, BLK), jnp.float32) for _ in range(R)]
            + [pltpu.SemaphoreType.DMA for _ in range(R + 1)]
        ),
        compiler_params=pltpu.CompilerParams(
            use_tc_tiling_on_sc=True, needs_layout_passes=False
        ),
    )
    out = gather(idx, wt)
    return out[:, :d_model].reshape(seq_len, batch, d_model)


# speedup vs baseline: 1.0037x; 1.0037x over previous
import math

import jax
import jax.numpy as jnp
from jax import lax
from jax.experimental import pallas as pl
from jax.experimental.pallas import tpu as pltpu
from jax.experimental.pallas import tpu_sc as plsc

D_MODEL = 64
SCALE = math.sqrt(D_MODEL)
OUT_W = 128

NC = 2
NS = 16
NW = NC * NS
L = 16

BLK = 128
R = 8


def _encoder_body(idx_hbm, wt_hbm, out_hbm, idx_v, out_v, *slabs_sems):
    slabs = slabs_sems[:R]
    sems = slabs_sems[R:2 * R]
    osem = slabs_sems[2 * R]
    n_per_w = idx_hbm.shape[1]
    wid = lax.axis_index("s") * NC + lax.axis_index("c")

    pltpu.sync_copy(idx_hbm.at[wid], idx_v.at[pl.ds(0, n_per_w)])

    iota = lax.iota(jnp.int32, L)

    def fire(t, r):
        blk0 = pl.multiple_of((t >> 7) << 7, BLK)
        pltpu.async_copy(
            wt_hbm.at[:, pl.ds(blk0, BLK)], slabs[r], sems[r]
        )

    def extract(t, i, r):
        j_vec = jnp.full((L,), t & (BLK - 1), jnp.int32)
        for c0 in range(0, D_MODEL, L):
            vals = plsc.load_gather(slabs[r], [c0 + iota, j_vec])
            out_v[i, pl.ds(c0, L)] = vals * SCALE

    t_prime = idx_v[pl.ds(0, L)]
    for r in range(R):
        fire(t_prime[r], r)

    def loop_body(g, carry):
        base = g * L
        t_cur = idx_v[pl.ds(base, L)]
        t_nxt = idx_v[pl.ds(base + L, L)]
        for k in range(L):
            i = base + k
            r = k % R
            pltpu.make_async_copy(
                wt_hbm.at[:, pl.ds(0, BLK)], slabs[r], sems[r]
            ).wait()
            extract(t_cur[k], i, r)
            t_ahead = t_cur[k + R] if k + R < L else t_nxt[k + R - L]

            @pl.when(i + R < n_per_w)
            def _():
                fire(t_ahead, r)

        return carry

    lax.fori_loop(0, n_per_w // L, loop_body, 0)

    pltpu.async_copy(
        out_v, out_hbm.at[pl.ds(wid * n_per_w, n_per_w)], osem
    ).wait()


def kernel(src, weight):
    seq_len, batch = src.shape
    n_tokens, d_model = weight.shape
    b_total = seq_len * batch
    assert d_model == D_MODEL
    n_per_w = b_total // NW
    assert b_total % NW == 0 and n_per_w % R == 0

    idx = src.reshape(NW, n_per_w).astype(jnp.int32)
    wt = weight.T

    gather = pl.kernel(
        _encoder_body,
        out_type=jax.ShapeDtypeStruct((b_total, OUT_W), jnp.float32),
        mesh=plsc.VectorSubcoreMesh(
            core_axis_name="c", subcore_axis_name="s",
            num_cores=NC, num_subcores=NS,
        ),
        scratch_types=(
            [
                pltpu.VMEM((n_per_w + L,), jnp.int32),
                pltpu.VMEM((n_per_w, OUT_W), jnp.float32),
            ]
            + [pltpu.VMEM((D_MODEL, BLK), jnp.float32) for _ in range(R)]
            + [pltpu.SemaphoreType.DMA for _ in range(R + 1)]
        ),
        compiler_params=pltpu.CompilerParams(
            use_tc_tiling_on_sc=True, needs_layout_passes=False
        ),
    )
    out = gather(idx, wt)
    return out[:, :d_model].reshape(seq_len, batch, d_model)
